# Initial kernel scaffold; baseline (speedup 1.0000x reference)
import jax
import jax.numpy as jnp
from jax.experimental import pallas as pl


def kernel(x, W1, b1, ln_gamma, ln_beta, W2, b2, temperature):
    raise NotImplementedError("write your pallas kernel here")


# speedup vs baseline: 2.5850x; 2.5850x over previous
import jax
import jax.numpy as jnp
from jax.experimental import pallas as pl
from jax.experimental.pallas import tpu as pltpu

_LN_EPS = 1e-5
_TOKEN_BLOCK = 512


def _gate_body(x_ref, w1_ref, b1_ref, g_ref, be_ref, w2_ref, b2_ref, it_ref,
               gates_ref, idx_ref, logits_ref):
    x = x_ref[...]
    h = jnp.dot(x, w1_ref[...], preferred_element_type=jnp.float32)
    h = h + b1_ref[...]
    mu = jnp.mean(h, axis=-1, keepdims=True)
    var = jnp.mean((h - mu) ** 2, axis=-1, keepdims=True)
    h = (h - mu) / jnp.sqrt(var + _LN_EPS) * g_ref[...] + be_ref[...]
    h = 0.5 * h * (1.0 + jax.lax.erf(h * 0.7071067811865476))
    logits = jnp.dot(h, w2_ref[...], preferred_element_type=jnp.float32)
    logits = (logits + b2_ref[...]) * it_ref[0, 0]

    e = logits.shape[-1]
    eid = jax.lax.broadcasted_iota(jnp.int32, logits.shape, 1)
    m0 = jnp.max(logits, axis=-1, keepdims=True)
    idx0 = jnp.min(jnp.where(logits == m0, eid, e), axis=-1, keepdims=True)
    masked = jnp.where(eid == idx0, -jnp.inf, logits)
    m1 = jnp.max(masked, axis=-1, keepdims=True)
    idx1 = jnp.min(jnp.where(masked == m1, eid, e), axis=-1, keepdims=True)

    e1 = jnp.exp(m1 - m0)
    scale = 1.0 / ((1.0 + e1) * (1.0 + 1e-10))
    g0 = scale
    g1 = e1 * scale
    gates = jnp.where(eid == idx0, g0, 0.0) + jnp.where(eid == idx1, g1, 0.0)

    gates_ref[...] = gates
    logits_ref[...] = logits
    idx_ref[...] = jnp.concatenate([idx0, idx1], axis=-1)


def kernel(x, W1, b1, ln_gamma, ln_beta, W2, b2, temperature):
    n, d = x.shape
    h = W1.shape[1]
    e = W2.shape[1]
    bt = min(_TOKEN_BLOCK, n)
    inv_t = (1.0 / jnp.clip(temperature, 0.5, 5.0)).reshape(1, 1)

    grid = (n // bt,)
    out_shape = (
        jax.ShapeDtypeStruct((n, e), jnp.float32),
        jax.ShapeDtypeStruct((n, 2), jnp.int32),
        jax.ShapeDtypeStruct((n, e), jnp.float32),
    )
    in_specs = [
        pl.BlockSpec((bt, d), lambda i: (i, 0)),
        pl.BlockSpec((d, h), lambda i: (0, 0)),
        pl.BlockSpec((1, h), lambda i: (0, 0)),
        pl.BlockSpec((1, h), lambda i: (0, 0)),
        pl.BlockSpec((1, h), lambda i: (0, 0)),
        pl.BlockSpec((h, e), lambda i: (0, 0)),
        pl.BlockSpec((1, e), lambda i: (0, 0)),
        pl.BlockSpec((1, 1), lambda i: (0, 0)),
    ]
    out_specs = (
        pl.BlockSpec((bt, e), lambda i: (i, 0)),
        pl.BlockSpec((bt, 2), lambda i: (i, 0)),
        pl.BlockSpec((bt, e), lambda i: (i, 0)),
    )
    gates, idx, logits = pl.pallas_call(
        _gate_body,
        grid=grid,
        in_specs=in_specs,
        out_specs=out_specs,
        out_shape=out_shape,
        compiler_params=pltpu.CompilerParams(
            dimension_semantics=("arbitrary",),
        ),
    )(x, W1, b1.reshape(1, h), ln_gamma.reshape(1, h), ln_beta.reshape(1, h),
      W2, b2.reshape(1, e), inv_t)
    return (gates, idx, logits)


# speedup vs baseline: 2.7283x; 1.0554x over previous
import jax
import jax.numpy as jnp
from jax.experimental import pallas as pl
from jax.experimental.pallas import tpu as pltpu

_LN_EPS = 1e-5
_TOKEN_BLOCK = 1024


def _gate_body(x_ref, w1_ref, b1_ref, g_ref, be_ref, w2_ref, b2_ref, it_ref,
               gates_ref, idx_ref, logits_ref):
    x = x_ref[...]
    h = jnp.dot(x, w1_ref[...], preferred_element_type=jnp.float32)
    h = h + b1_ref[...]
    mu = jnp.mean(h, axis=-1, keepdims=True)
    var = jnp.mean((h - mu) ** 2, axis=-1, keepdims=True)
    h = (h - mu) / jnp.sqrt(var + _LN_EPS) * g_ref[...] + be_ref[...]
    h = 0.5 * h * (1.0 + jax.lax.erf(h * 0.7071067811865476))
    logits = jnp.dot(h, w2_ref[...], preferred_element_type=jnp.float32)
    logits = (logits + b2_ref[...]) * it_ref[0, 0]

    e = logits.shape[-1]
    eid = jax.lax.broadcasted_iota(jnp.int32, logits.shape, 1)
    m0 = jnp.max(logits, axis=-1, keepdims=True)
    idx0 = jnp.min(jnp.where(logits == m0, eid, e), axis=-1, keepdims=True)
    masked = jnp.where(eid == idx0, -jnp.inf, logits)
    m1 = jnp.max(masked, axis=-1, keepdims=True)
    idx1 = jnp.min(jnp.where(masked == m1, eid, e), axis=-1, keepdims=True)

    e1 = jnp.exp(m1 - m0)
    scale = 1.0 / ((1.0 + e1) * (1.0 + 1e-10))
    g0 = scale
    g1 = e1 * scale
    gates = jnp.where(eid == idx0, g0, 0.0) + jnp.where(eid == idx1, g1, 0.0)

    gates_ref[...] = gates
    logits_ref[...] = logits
    idx_ref[...] = jnp.concatenate([idx0, idx1], axis=-1)


def kernel(x, W1, b1, ln_gamma, ln_beta, W2, b2, temperature):
    n, d = x.shape
    h = W1.shape[1]
    e = W2.shape[1]
    bt = min(_TOKEN_BLOCK, n)
    inv_t = (1.0 / jnp.clip(temperature, 0.5, 5.0)).reshape(1, 1)

    grid = (n // bt,)
    out_shape = (
        jax.ShapeDtypeStruct((n, e), jnp.float32),
        jax.ShapeDtypeStruct((n, 2), jnp.int32),
        jax.ShapeDtypeStruct((n, e), jnp.float32),
    )
    in_specs = [
        pl.BlockSpec((bt, d), lambda i: (i, 0)),
        pl.BlockSpec((d, h), lambda i: (0, 0)),
        pl.BlockSpec((1, h), lambda i: (0, 0)),
        pl.BlockSpec((1, h), lambda i: (0, 0)),
        pl.BlockSpec((1, h), lambda i: (0, 0)),
        pl.BlockSpec((h, e), lambda i: (0, 0)),
        pl.BlockSpec((1, e), lambda i: (0, 0)),
        pl.BlockSpec((1, 1), lambda i: (0, 0)),
    ]
    out_specs = (
        pl.BlockSpec((bt, e), lambda i: (i, 0)),
        pl.BlockSpec((bt, 2), lambda i: (i, 0)),
        pl.BlockSpec((bt, e), lambda i: (i, 0)),
    )
    gates, idx, logits = pl.pallas_call(
        _gate_body,
        grid=grid,
        in_specs=in_specs,
        out_specs=out_specs,
        out_shape=out_shape,
        compiler_params=pltpu.CompilerParams(
            dimension_semantics=("arbitrary",),
        ),
    )(x, W1, b1.reshape(1, h), ln_gamma.reshape(1, h), ln_beta.reshape(1, h),
      W2, b2.reshape(1, e), inv_t)
    return (gates, idx, logits)
